# Initial kernel scaffold; baseline (speedup 1.0000x reference)
import jax
import jax.numpy as jnp
from jax.experimental import pallas as pl


def kernel(params, agent_cls, agent_obs_xy, agent_futs_xy, nbrs_obs_xy, nbrs_obs_padding, seq_ids, nbrs_start_end_idx, cls_start_end_idx, trajs_start_end_idx):
    raise NotImplementedError("write your pallas kernel here")


# speedup vs baseline: 7.3632x; 7.3632x over previous
import jax
import jax.numpy as jnp
from jax.experimental import pallas as pl
from jax.experimental.pallas import tpu as pltpu

_B = 16
_SLABS = 1
_SB = _B // _SLABS
_L = 8
_PP = 8
_NN = 64
_P = 64
_TOBS = 20
_TFUT = 30
_PREDS = _L * _PP

_F32 = jnp.float32


def _leaky(x):
    return jnp.where(x >= 0, x, 0.1 * x)


def _mm(a, b):
    return jax.lax.dot_general(a, b, (((1,), (0,)), ((), ())),
                               preferred_element_type=_F32)


def _nt(a, b):
    return jax.lax.dot_general(a, b, (((1,), (1,)), ((), ())),
                               preferred_element_type=_F32)


def _tn(a, b):
    return jax.lax.dot_general(a, b, (((0,), (0,)), ((), ())),
                               preferred_element_type=_F32)


def _iota(shape, dim):
    return jax.lax.broadcasted_iota(jnp.int32, shape, dim)


def kernel(params, agent_cls, agent_obs_xy, agent_futs_xy, nbrs_obs_xy,
           nbrs_obs_padding, seq_ids, nbrs_start_end_idx, cls_start_end_idx,
           trajs_start_end_idx):
    f32 = _F32
    p = params

    names, ops, specs = [], [], []

    def add_data(name, arr):
        names.append(name)
        ops.append(arr.astype(f32))
        shape = arr.shape
        specs.append(pl.BlockSpec(shape, lambda i: (0, 0, 0)))

    def add_w(name, arr):
        arr = arr.astype(f32)
        names.append(name)
        ops.append(arr)
        specs.append(pl.BlockSpec(arr.shape, lambda i: (0, 0)))

    add_data('cls_x0', agent_cls[..., 0].reshape(_SLABS, 1, _SB * _L * _P))
    add_data('cls_x1', agent_cls[..., 1].reshape(_SLABS, 1, _SB * _L * _P))

    def tmaj(a, rows, t):
        return (a.reshape(_SLABS, _SB * rows, t).transpose(0, 2, 1)
                .reshape(_SLABS, t * _SB * rows, 1))

    def tpack(arrs, rows, t):
        tp = -(-t // 4) * 4
        cols = []
        for a in arrs:
            a = jnp.pad(a, ((0, 0), (0, tp - t)))
            cols.append(a.reshape(_SLABS, _SB * rows, tp // 4, 4)
                        .transpose(0, 2, 1, 3)
                        .reshape(_SLABS, (tp // 4) * _SB * rows, 4))
        return jnp.concatenate(cols, axis=-1)

    add_data('nbr_X', tpack([nbrs_obs_xy[..., 0], nbrs_obs_xy[..., 1],
                             nbrs_obs_padding], _NN, _TOBS))
    add_data('fut_X', tpack([agent_futs_xy[..., 0], agent_futs_xy[..., 1]],
                            _PREDS, _TFUT))
    add_data('obs_x0', tmaj(agent_obs_xy[..., 0], 1, _TOBS))
    add_data('obs_x1', tmaj(agent_obs_xy[..., 1], 1, _TOBS))

    def add_enc(tag, enc, map_lin, n_coord):
        we = enc['emb']['w']
        for c in range(n_coord):
            add_w(f'{tag}_we{c}', we[c:c + 1])
        add_w(f'{tag}_web', enc['emb']['b'].reshape(1, -1))
        add_w(f'{tag}_wenc', enc['enc']['w'])
        add_w(f'{tag}_wencb', enc['enc']['b'].reshape(1, -1))
        add_w(f'{tag}_wmap', map_lin['w'])
        add_w(f'{tag}_wmapb', map_lin['b'].reshape(1, -1))

    ce_ = p['cls_encoder']
    add_w('cls_we0c', ce_['emb']['w'][0:1].T)
    add_w('cls_we1c', ce_['emb']['w'][1:2].T)
    add_w('cls_webc', ce_['emb']['b'].reshape(-1, 1))
    add_w('cls_wencT', ce_['enc']['w'].T)
    add_w('cls_encbc', ce_['enc']['b'].reshape(-1, 1))
    add_w('cls_wmapT', p['cls_map']['w'].T)
    add_w('cls_mapbc', p['cls_map']['b'].reshape(-1, 1))
    add_enc('obs', p['obs_encoder'], p['obs_map'], 2)

    def add_packed(tag, enc, map_lin, n_coord, t):
        we = enc['emb']['w']
        eye4 = jnp.eye(4, dtype=f32)
        add_w(f'{tag}_Wp', jnp.concatenate(
            [jnp.kron(eye4, we[c:c + 1]) for c in range(n_coord)], axis=0))
        web = enc['emb']['b'].reshape(1, -1)
        add_w(f'{tag}_bp', jnp.tile(web, (1, 4)))
        wenc = enc['enc']['w']
        add_w(f'{tag}_wenc4', jnp.kron(jnp.ones((4, 1), f32), wenc))
        tp = -(-t // 4) * 4
        lw = jnp.where(web >= 0, web, 0.1 * web)
        adj = (enc['enc']['b'].reshape(1, -1)
               - ((tp - t) / t) * (lw @ wenc))
        add_w(f'{tag}_wencb', adj)
        add_w(f'{tag}_wmap', map_lin['w'])
        add_w(f'{tag}_wmapb', map_lin['b'].reshape(1, -1))

    add_packed('nbr', p['nbrs_encoder'], p['nbrs_map'], 3, _TOBS)
    add_packed('prd', p['pred_encoder'], p['pred_map'], 2, _TFUT)

    for tag in ('l2a', 'a2a', 'l2f', 'f2f'):
        add_w(f'{tag}_w', p[tag]['out']['w'])
        add_w(f'{tag}_b', p[tag]['out']['b'].reshape(1, -1))
    add_w('soc_w', p['social_map']['w'])
    add_w('soc_b', p['social_map']['b'].reshape(1, -1))
    add_w('ap_w', p['allpreds_map']['w'])
    add_w('ap_b', p['allpreds_map']['b'].reshape(1, -1))

    ld = p['lane_dec']
    add_w('ld_w1a', ld['fc1']['w'][0:64])
    add_w('ld_w1b', ld['fc1']['w'][64:128])
    add_w('ld_b1', ld['fc1']['b'].reshape(1, -1))
    add_w('ld_w2', ld['fc2']['w'])
    add_w('ld_b2', ld['fc2']['b'].reshape(1, -1))
    add_w('ld_w3', ld['fc3']['w'])
    add_w('ld_b3', ld['fc3']['b'].reshape(1, -1))
    td = p['traj_dec']
    add_w('td_w1a', td['fc1']['w'][0:64])
    add_w('td_w1b', td['fc1']['w'][64:128])
    add_w('td_w1c', td['fc1']['w'][128:256])
    add_w('td_b1', td['fc1']['b'].reshape(1, -1))
    add_w('td_w2', td['fc2']['w'])
    add_w('td_b2', td['fc2']['b'].reshape(1, -1))
    add_w('td_w3', td['fc3']['w'])
    add_w('td_b3', td['fc3']['b'].reshape(1, -1))

    n_in = len(names)
    _NR = _SB * _NN
    _PR = _SB * _PREDS

    def body(*refs):
        gref = {nm: refs[k] for k, nm in enumerate(names)}
        g = {nm: gref[nm][...] for nm in names
             if not nm.endswith(('x0', 'x1', '_X'))}
        lane_ref = refs[n_in]
        traj_ref = refs[n_in + 1]
        obs_scr = refs[n_in + 2]
        int_scr = refs[n_in + 3]
        alp_scr = refs[n_in + 4]

        Bsel = (_iota((512, 8), 0) // 64 == _iota((512, 8), 1)).astype(f32)
        Bmask8 = (_iota((8, 512), 1) // 64 == _iota((8, 512), 0)).astype(f32)
        negC = jnp.where(_iota((64, 512), 1) // 64
                         == _iota((64, 512), 0) // _PP,
                         0.0, -1e30).astype(f32)
        eye8 = (_iota((8, 8), 0) == _iota((8, 8), 1)).astype(f32)
        ones18 = jnp.ones((1, 8), f32)

        for sl in range(_SLABS):
            hT = _leaky(g['cls_we0c'] * gref['cls_x0'][sl]
                        + g['cls_we1c'] * gref['cls_x1'][sl]
                        + g['cls_webc'])
            CET_all = _mm(g['cls_wencT'], hT) + g['cls_encbc']
            CMT_all = _mm(g['cls_wmapT'], CET_all) + g['cls_mapbc']

            h = _leaky(_mm(gref['nbr_X'][sl], g['nbr_Wp']) + g['nbr_bp'])
            s = h[0:_NR]
            for t in range(1, _TOBS // 4):
                s = s + h[t * _NR:(t + 1) * _NR]
            ne_all = _mm(s, g['nbr_wenc4']) * (1.0 / _TOBS) + g['nbr_wencb']
            nm_all = _mm(ne_all, g['nbr_wmap']) + g['nbr_wmapb']

            h = _leaky(_mm(gref['fut_X'][sl], g['prd_Wp']) + g['prd_bp'])
            s = h[0:_PR]
            for t in range(1, _TFUT // 4 + 1):
                s = s + h[t * _PR:(t + 1) * _PR]
            pe_all = _mm(s, g['prd_wenc4']) * (1.0 / _TFUT) + g['prd_wencb']
            pm_all = _mm(pe_all, g['prd_wmap']) + g['prd_wmapb']

            h = _leaky(gref['obs_x0'][sl] * g['obs_we0'] + gref['obs_x1'][sl] * g['obs_we1']
                       + g['obs_web'])
            s = h[0:_SB]
            for t in range(1, _TOBS):
                s = s + h[t * _SB:(t + 1) * _SB]
            oe_all = _mm(s * (1.0 / _TOBS), g['obs_wenc']) + g['obs_wencb']
            om_all = _mm(oe_all, g['obs_wmap']) + g['obs_wmapb']

            wl2a, bl2a = g['l2a_w'], g['l2a_b']
            wsoc, bsoc = g['soc_w'], g['soc_b']
            wa2a, ba2a = g['a2a_w'], g['a2a_b']
            wl2f, bl2f = g['l2f_w'], g['l2f_b']

            rng = range(_SB)
            CETs = [CET_all[:, s * 512:(s + 1) * 512] for s in rng]
            CMTs = [CMT_all[:, s * 512:(s + 1) * 512] for s in rng]
            nes = [ne_all[s * 64:(s + 1) * 64] for s in rng]
            nms = [nm_all[s * 64:(s + 1) * 64] for s in rng]
            pes = [pe_all[s * 64:(s + 1) * 64] for s in rng]
            pms = [pm_all[s * 64:(s + 1) * 64] for s in rng]
            ohs = [(_iota((1, _SB), 1) == s).astype(f32) for s in rng]
            oes = [_mm(ohs[s], oe_all) for s in rng]
            oms = [_mm(ohs[s], om_all) for s in rng]

            Sns = [_mm(nms[s], CMTs[s]) * 0.125 for s in rng]
            Sos = [_mm(oms[s], CMTs[s]) * 0.125 for s in rng]
            S3s = [_mm(pms[s], CMTs[s]) * 0.125 + negC for s in rng]
            Ens = [jnp.exp(x - jnp.max(x, axis=-1, keepdims=True)) for x in Sns]
            Eos = [jnp.exp(x - jnp.max(x, axis=-1, keepdims=True)) for x in Sos]
            E3s = [jnp.exp(x - jnp.max(x, axis=-1, keepdims=True)) for x in S3s]
            Dns = [_mm(x, Bsel) for x in Ens]
            DoTs = [_nt(eye8, _mm(x, Bsel)) for x in Eos]

            attOs = [_nt(Eos[s] * Bmask8, CETs[s]) / DoTs[s] for s in rng]
            att3s = [_nt(E3s[s], CETs[s])
                     / jnp.sum(E3s[s], axis=-1, keepdims=True) for s in rng]
            aol_o8s = [oes[s] + _mm(attOs[s], wl2a) + bl2a for s in rng]
            pred_lanes = [pes[s] + _mm(att3s[s], wl2f) + bl2f for s in rng]
            alm_o8s = [_mm(x, wsoc) + bsoc for x in aol_o8s]

            yhat8s = [_nt(x, wsoc) for x in alm_o8s]
            zhat8s = [_nt(x, wl2a) for x in yhat8s]
            VZbTs = [_mm(zhat8s[s], CETs[s]) * Bmask8 for s in rng]
            S2ns = [(_nt(Ens[s], VZbTs[s]) / Dns[s] + _nt(nes[s], yhat8s[s])
                     + _nt(bl2a, yhat8s[s]) + _nt(bsoc, alm_o8s[s])) * 0.125
                    for s in rng]
            Gs = [_nt(x, x) for x in alm_o8s]
            s2os = [_mm(ones18, Gs[s] * eye8) * 0.125 for s in rng]
            m2s = [jnp.maximum(jnp.max(S2ns[s], axis=0, keepdims=True), s2os[s])
                   for s in rng]
            ens = [jnp.exp(S2ns[s] - m2s[s]) for s in rng]
            eos = [jnp.exp(s2os[s] - m2s[s]) for s in rng]
            csums = [jnp.sum(x, axis=0, keepdims=True) for x in ens]

            plms = [_mm(x, g['ap_w']) + g['ap_b'] for x in pred_lanes]
            S4s = [_nt(x, x) * 0.125 for x in plms]
            E4s = [jnp.exp(x - jnp.max(x, axis=-1, keepdims=True)) for x in S4s]
            att4s = [_mm(E4s[s], pred_lanes[s])
                     / jnp.sum(E4s[s], axis=-1, keepdims=True) for s in rng]
            allps = [pred_lanes[s] + _mm(att4s[s], g['f2f_w']) + g['f2f_b']
                     for s in rng]

            EN_nes = [_tn(ens[s], nes[s]) for s in rng]
            W8bs = [_tn(ens[s] / Dns[s], Ens[s]) * Bmask8 for s in rng]
            Us = [_nt(W8bs[s], CETs[s]) for s in rng]
            csumTs = [_nt(eye8, x) for x in csums]
            eoTs = [_nt(eye8, x) for x in eos]
            denTs = [_nt(eye8, csums[s] + eos[s]) for s in rng]
            att2s = [(EN_nes[s] + _mm(Us[s], wl2a) + csumTs[s] * bl2a
                      + eoTs[s] * aol_o8s[s]) / denTs[s] for s in rng]
            int8s = [aol_o8s[s] + _mm(att2s[s], wa2a) + ba2a for s in rng]

            for s in rng:
                obs_scr[s * _L:(s + 1) * _L, :] = aol_o8s[s]
                int_scr[s * _L:(s + 1) * _L, :] = int8s[s]
                alp_scr[s * _PREDS:(s + 1) * _PREDS, :] = allps[s]

            obs_all = obs_scr[...]
            int_all = int_scr[...]
            allp_all = alp_scr[...]

            h1 = _leaky(_mm(obs_all, g['ld_w1a']) + _mm(int_all, g['ld_w1b'])
                        + g['ld_b1'])
            h2 = _leaky(_mm(h1, g['ld_w2']) + g['ld_b2'])
            lane_ref[sl * _SB * _L:(sl + 1) * _SB * _L, :] = (
                _mm(h2, g['ld_w3']) + g['ld_b3'])

            u_all = _mm(obs_all, g['td_w1a']) + _mm(int_all, g['td_w1b'])
            rep_u = jnp.repeat(u_all, _PP, axis=0)
            t1 = _leaky(rep_u + _mm(allp_all, g['td_w1c'])
                        + g['td_b1'])
            t2 = _leaky(_mm(t1, g['td_w2']) + g['td_b2'])
            traj_ref[sl * _PR:(sl + 1) * _PR, :] = (
                _mm(t2, g['td_w3']) + g['td_b3'])

    out = pl.pallas_call(
        body,
        grid=(1,),
        in_specs=specs,
        out_specs=[
            pl.BlockSpec((_B * _L, 1), lambda i: (0, 0)),
            pl.BlockSpec((_B * _PREDS, 1), lambda i: (0, 0)),
        ],
        out_shape=[
            jax.ShapeDtypeStruct((_B * _L, 1), f32),
            jax.ShapeDtypeStruct((_B * _PREDS, 1), f32),
        ],
        scratch_shapes=[
            pltpu.VMEM((_SB * _L, 64), f32),
            pltpu.VMEM((_SB * _L, 64), f32),
            pltpu.VMEM((_SB * _PREDS, 128), f32),
        ],
    )(*ops)
    return out[0], out[1]
